# baseline (speedup 1.0000x reference)
import functools

import jax
import jax.numpy as jnp
from jax import lax
from jax.experimental import pallas as pl
from jax.experimental.pallas import tpu as pltpu
from jax.experimental.pallas import tpu_sc as plsc

L = 16
NS = 16
N_ITERS = 50


def _tc_util_body(w_ref, b_ref, feats_ref, util_ref, m_ref):
    f = feats_ref[...]
    w = w_ref[...]
    u = lax.dot_general(w, f, (((1,), (1,)), ((), ())),
                        preferred_element_type=jnp.float32)
    u = u + b_ref[0]
    u = jnp.clip(u, -100.0, -1e-06)
    util_ref[...] = u.reshape(u.shape[1])
    m_ref[...] = jnp.exp(u).reshape(u.shape[1])


def _tc_util(feats, W, b):
    E, F = feats.shape
    BE = 512
    NB = E // BE
    return pl.pallas_call(
        _tc_util_body,
        grid=(NB,),
        in_specs=[
            pl.BlockSpec((1, F), lambda i: (0, 0)),
            pl.BlockSpec(memory_space=pltpu.SMEM),
            pl.BlockSpec((BE, F), lambda i: (i, 0)),
        ],
        out_specs=[
            pl.BlockSpec((BE,), lambda i: (i,)),
            pl.BlockSpec((BE,), lambda i: (i,)),
        ],
        out_shape=[
            jax.ShapeDtypeStruct((E,), jnp.float32),
            jax.ShapeDtypeStruct((E,), jnp.float32),
        ],
    )(W, b, feats)


def _tc_log_body(z_ref, v_ref):
    v_ref[...] = jnp.log(jnp.maximum(z_ref[...], 1e-30))


def _tc_log(z2d):
    return pl.pallas_call(
        _tc_log_body,
        out_shape=jax.ShapeDtypeStruct(z2d.shape, jnp.float32),
    )(z2d)


def _sc_body(NP, EPW, m_hbm, src_hbm, dst_hbm, bv_hbm, z_hbm, prob_hbm,
             m_v, pk_v, dst_v, z_v, z2_v, bv_v, part_v, red_v, zs_v,
             sp_part, sp_z):
    SL = NP // NS
    sid = lax.axis_index("s")
    base = sid * EPW

    pltpu.sync_copy(m_hbm.at[pl.ds(base, EPW)], m_v)
    pltpu.sync_copy(src_hbm.at[pl.ds(base, EPW)], pk_v)
    pltpu.sync_copy(dst_hbm.at[pl.ds(base, EPW)], dst_v)
    pltpu.sync_copy(bv_hbm, bv_v)
    pltpu.sync_copy(bv_hbm, z_v)

    @plsc.parallel_loop(0, EPW, L, unroll=8)
    def _(off):
        s = pk_v[pl.ds(off, L)]
        d = dst_v[pl.ds(off, L)]
        pk_v[pl.ds(off, L)] = lax.bitwise_or(lax.shift_left(s, 16), d)

    zero = jnp.zeros((L,), jnp.float32)

    def one_iter(z_src, z_dst):
        @plsc.parallel_loop(0, NP, L, unroll=8)
        def _(off):
            part_v[pl.ds(off, L)] = zero

        @plsc.parallel_loop(0, EPW, L, unroll=10)
        def _(off):
            pk = pk_v[pl.ds(off, L)]
            d = lax.bitwise_and(pk, 0xFFFF)
            s = lax.shift_right_logical(pk, 16)
            zg = plsc.load_gather(z_src, [d])
            msg = m_v[pl.ds(off, L)] * zg
            plsc.addupdate_scatter(part_v, [s], msg)

        pltpu.sync_copy(part_v, sp_part.at[sid])
        plsc.subcore_barrier()
        pltpu.sync_copy(sp_part.at[:, pl.ds(sid * SL, SL)], red_v)

        @plsc.parallel_loop(0, SL, L, unroll=4)
        def _(i):
            acc = bv_v[pl.ds(sid * SL + i, L)]
            for j in range(NS):
                acc = acc + red_v[j, pl.ds(i, L)]
            zs_v[pl.ds(i, L)] = acc

        pltpu.sync_copy(zs_v, sp_z.at[pl.ds(sid * SL, SL)])
        plsc.subcore_barrier()
        pltpu.sync_copy(sp_z, z_dst)

    def iter_cond(carry):
        t, conv = carry
        return jnp.logical_and(t < N_ITERS, conv == 0)

    def iter_body(carry):
        t, conv = carry
        one_iter(z_v, z2_v)
        one_iter(z2_v, z_v)

        def neq_body(i, acc):
            return acc + (z_v[pl.ds(i * L, L)] != z2_v[pl.ds(i * L, L)]
                          ).astype(jnp.float32)

        neq = lax.fori_loop(0, NP // L, neq_body, jnp.zeros((L,), jnp.float32))
        nz = jnp.max(neq)
        conv = jnp.where(nz == 0.0, 1, 0).astype(jnp.int32)
        return t + 2, conv

    lax.while_loop(iter_cond, iter_body, (jnp.int32(0), jnp.int32(0)))

    @plsc.parallel_loop(0, NP, L, unroll=8)
    def _(off):
        z_v[pl.ds(off, L)] = jnp.maximum(z_v[pl.ds(off, L)], 1e-30)
    pltpu.sync_copy(z_v.at[pl.ds(sid * SL, SL)], z_hbm.at[pl.ds(sid * SL, SL)])

    @plsc.parallel_loop(0, EPW, L, unroll=8)
    def _(off):
        pk = pk_v[pl.ds(off, L)]
        d = lax.bitwise_and(pk, 0xFFFF)
        s = lax.shift_right_logical(pk, 16)
        zd = plsc.load_gather(z_v, [d])
        zs = plsc.load_gather(z_v, [s])
        m_v[pl.ds(off, L)] = m_v[pl.ds(off, L)] * zd / zs
    pltpu.sync_copy(m_v, prob_hbm.at[pl.ds(base, EPW)])


def _sc_fixed_point(m, src, dst, bfull):
    E, = m.shape
    NP, = bfull.shape
    EPW = E // NS
    SL = NP // NS
    mesh = plsc.VectorSubcoreMesh(core_axis_name="c", subcore_axis_name="s",
                                  num_cores=1)
    f = pl.kernel(
        functools.partial(_sc_body, NP, EPW),
        out_type=[
            jax.ShapeDtypeStruct((NP,), jnp.float32),
            jax.ShapeDtypeStruct((E,), jnp.float32),
        ],
        mesh=mesh,
        compiler_params=pltpu.CompilerParams(needs_layout_passes=False),
        scratch_types=[
            pltpu.VMEM((EPW,), jnp.float32),
            pltpu.VMEM((EPW,), jnp.int32),
            pltpu.VMEM((EPW,), jnp.int32),
            pltpu.VMEM((NP,), jnp.float32),
            pltpu.VMEM((NP,), jnp.float32),
            pltpu.VMEM((NP,), jnp.float32),
            pltpu.VMEM((NP,), jnp.float32),
            pltpu.VMEM((NS, SL), jnp.float32),
            pltpu.VMEM((SL,), jnp.float32),
            pltpu.VMEM_SHARED((NS, NP), jnp.float32),
            pltpu.VMEM_SHARED((NP,), jnp.float32),
        ],
    )
    return f(m, src, dst, bfull)


def kernel(feats, dest_mask, edge_index, batch, n_nodes, W, b):
    E, F = feats.shape
    N = dest_mask.shape[0]
    NP = ((N + NS * L - 1) // (NS * L)) * (NS * L)
    if NP % 128 != 0:
        NP = ((NP + 127) // 128) * 128

    util1, m = _tc_util(feats, W, b)
    util = util1.reshape(E, 1)

    src = edge_index[0]
    dst = edge_index[1]
    bfull = jnp.zeros((NP,), jnp.float32).at[:N].set(
        dest_mask.astype(jnp.float32))

    zc, prob = _sc_fixed_point(m, src, dst, bfull)

    value = _tc_log(zc.reshape(NP // 128, 128)).reshape(NP)[:N]
    return value, util, prob


# speedup vs baseline: 3.2848x; 3.2848x over previous
import functools

import jax
import jax.numpy as jnp
from jax import lax
from jax.experimental import pallas as pl
from jax.experimental.pallas import tpu as pltpu
from jax.experimental.pallas import tpu_sc as plsc

L = 16
NS = 16
N_ITERS = 50


def _tc_util_body(w_ref, b_ref, feats_ref, util_ref, m_ref):
    f = feats_ref[...]
    w = w_ref[...]
    u = lax.dot_general(w, f, (((1,), (1,)), ((), ())),
                        preferred_element_type=jnp.float32)
    u = u + b_ref[0]
    u = jnp.clip(u, -100.0, -1e-06)
    util_ref[...] = u[None]
    m_ref[...] = jnp.exp(u)[None]


def _tc_util(feats, W, b):
    E, F = feats.shape
    BE = 8000
    NB = E // BE
    return pl.pallas_call(
        _tc_util_body,
        grid=(NB,),
        in_specs=[
            pl.BlockSpec((1, F), lambda i: (0, 0)),
            pl.BlockSpec(memory_space=pltpu.SMEM),
            pl.BlockSpec((BE, F), lambda i: (i, 0)),
        ],
        out_specs=[
            pl.BlockSpec((1, 1, BE), lambda i: (i, 0, 0)),
            pl.BlockSpec((1, 1, BE), lambda i: (i, 0, 0)),
        ],
        out_shape=[
            jax.ShapeDtypeStruct((NB, 1, BE), jnp.float32),
            jax.ShapeDtypeStruct((NB, 1, BE), jnp.float32),
        ],
    )(W, b, feats)


def _tc_log_body(z_ref, v_ref):
    v_ref[...] = jnp.log(jnp.maximum(z_ref[...], 1e-30))


def _tc_log(z2d):
    return pl.pallas_call(
        _tc_log_body,
        out_shape=jax.ShapeDtypeStruct(z2d.shape, jnp.float32),
    )(z2d)


def _sc_body(NP, EPW, m_hbm, src_hbm, dst_hbm, bv_hbm, z_hbm, prob_hbm,
             m_v, pk_v, dst_v, z_v, z2_v, bv_v, part_v, red_v, zs_v,
             sp_part, sp_z):
    SL = NP // NS
    sid = lax.axis_index("s")
    base = sid * EPW

    pltpu.sync_copy(m_hbm.at[pl.ds(base, EPW)], m_v)
    pltpu.sync_copy(src_hbm.at[pl.ds(base, EPW)], pk_v)
    pltpu.sync_copy(dst_hbm.at[pl.ds(base, EPW)], dst_v)
    pltpu.sync_copy(bv_hbm, bv_v)
    pltpu.sync_copy(bv_hbm, z_v)

    @plsc.parallel_loop(0, EPW, L, unroll=8)
    def _(off):
        s = pk_v[pl.ds(off, L)]
        d = dst_v[pl.ds(off, L)]
        pk_v[pl.ds(off, L)] = lax.bitwise_or(lax.shift_left(s, 16), d)

    zero = jnp.zeros((L,), jnp.float32)

    def one_iter(z_src, z_dst):
        @plsc.parallel_loop(0, NP, L, unroll=8)
        def _(off):
            part_v[pl.ds(off, L)] = zero

        @plsc.parallel_loop(0, EPW, L, unroll=10)
        def _(off):
            pk = pk_v[pl.ds(off, L)]
            d = lax.bitwise_and(pk, 0xFFFF)
            s = lax.shift_right_logical(pk, 16)
            zg = plsc.load_gather(z_src, [d])
            msg = m_v[pl.ds(off, L)] * zg
            plsc.addupdate_scatter(part_v, [s], msg)

        pltpu.sync_copy(part_v, sp_part.at[sid])
        plsc.subcore_barrier()
        pltpu.sync_copy(sp_part.at[:, pl.ds(sid * SL, SL)], red_v)

        @plsc.parallel_loop(0, SL, L, unroll=4)
        def _(i):
            acc = bv_v[pl.ds(sid * SL + i, L)]
            for j in range(NS):
                acc = acc + red_v[j, pl.ds(i, L)]
            zs_v[pl.ds(i, L)] = acc

        pltpu.sync_copy(zs_v, sp_z.at[pl.ds(sid * SL, SL)])
        plsc.subcore_barrier()
        pltpu.sync_copy(sp_z, z_dst)

    def iter_cond(carry):
        t, conv = carry
        return jnp.logical_and(t < N_ITERS, conv == 0)

    def iter_body(carry):
        t, conv = carry
        one_iter(z_v, z2_v)
        one_iter(z2_v, z_v)

        def neq_body(i, acc):
            return acc + (z_v[pl.ds(i * L, L)] != z2_v[pl.ds(i * L, L)]
                          ).astype(jnp.float32)

        neq = lax.fori_loop(0, NP // L, neq_body, jnp.zeros((L,), jnp.float32))
        nz = jnp.max(neq)
        conv = jnp.where(nz == 0.0, 1, 0).astype(jnp.int32)
        return t + 2, conv

    lax.while_loop(iter_cond, iter_body, (jnp.int32(0), jnp.int32(0)))

    @plsc.parallel_loop(0, NP, L, unroll=8)
    def _(off):
        z_v[pl.ds(off, L)] = jnp.maximum(z_v[pl.ds(off, L)], 1e-30)
    pltpu.sync_copy(z_v.at[pl.ds(sid * SL, SL)], z_hbm.at[pl.ds(sid * SL, SL)])

    @plsc.parallel_loop(0, EPW, L, unroll=8)
    def _(off):
        pk = pk_v[pl.ds(off, L)]
        d = lax.bitwise_and(pk, 0xFFFF)
        s = lax.shift_right_logical(pk, 16)
        zd = plsc.load_gather(z_v, [d])
        zs = plsc.load_gather(z_v, [s])
        m_v[pl.ds(off, L)] = m_v[pl.ds(off, L)] * zd / zs
    pltpu.sync_copy(m_v, prob_hbm.at[pl.ds(base, EPW)])


def _sc_fixed_point(m, src, dst, bfull):
    E, = m.shape
    NP, = bfull.shape
    EPW = E // NS
    SL = NP // NS
    mesh = plsc.VectorSubcoreMesh(core_axis_name="c", subcore_axis_name="s",
                                  num_cores=1)
    f = pl.kernel(
        functools.partial(_sc_body, NP, EPW),
        out_type=[
            jax.ShapeDtypeStruct((NP,), jnp.float32),
            jax.ShapeDtypeStruct((E,), jnp.float32),
        ],
        mesh=mesh,
        compiler_params=pltpu.CompilerParams(needs_layout_passes=False),
        scratch_types=[
            pltpu.VMEM((EPW,), jnp.float32),
            pltpu.VMEM((EPW,), jnp.int32),
            pltpu.VMEM((EPW,), jnp.int32),
            pltpu.VMEM((NP,), jnp.float32),
            pltpu.VMEM((NP,), jnp.float32),
            pltpu.VMEM((NP,), jnp.float32),
            pltpu.VMEM((NP,), jnp.float32),
            pltpu.VMEM((NS, SL), jnp.float32),
            pltpu.VMEM((SL,), jnp.float32),
            pltpu.VMEM_SHARED((NS, NP), jnp.float32),
            pltpu.VMEM_SHARED((NP,), jnp.float32),
        ],
    )
    return f(m, src, dst, bfull)


def kernel(feats, dest_mask, edge_index, batch, n_nodes, W, b):
    E, F = feats.shape
    N = dest_mask.shape[0]
    NP = ((N + NS * L - 1) // (NS * L)) * (NS * L)
    if NP % 128 != 0:
        NP = ((NP + 127) // 128) * 128

    util2, m2 = _tc_util(feats, W, b)
    util = util2.reshape(E, 1)
    m = m2.reshape(E)

    src = edge_index[0]
    dst = edge_index[1]
    bfull = jnp.zeros((NP,), jnp.float32).at[:N].set(
        dest_mask.astype(jnp.float32))

    zc, prob = _sc_fixed_point(m, src, dst, bfull)

    value = _tc_log(zc.reshape(NP // 128, 128)).reshape(NP)[:N]
    return value, util, prob


# speedup vs baseline: 3.6534x; 1.1122x over previous
import functools

import jax
import jax.numpy as jnp
from jax import lax
from jax.experimental import pallas as pl
from jax.experimental.pallas import tpu as pltpu
from jax.experimental.pallas import tpu_sc as plsc

L = 16
NS = 16
N_ITERS = 50


def _tc_util_body(w_ref, b_ref, feats_ref, util_ref, m_ref):
    f = feats_ref[...]
    w = w_ref[...]
    u = lax.dot_general(w, f, (((1,), (1,)), ((), ())),
                        preferred_element_type=jnp.float32)
    u = u + b_ref[0]
    u = jnp.clip(u, -100.0, -1e-06)
    util_ref[...] = u[None]
    m_ref[...] = jnp.exp(u)[None]


def _tc_util(feats, W, b):
    E, F = feats.shape
    BE = 16000
    NB = E // BE
    return pl.pallas_call(
        _tc_util_body,
        grid=(NB,),
        in_specs=[
            pl.BlockSpec((1, F), lambda i: (0, 0)),
            pl.BlockSpec(memory_space=pltpu.SMEM),
            pl.BlockSpec((BE, F), lambda i: (i, 0)),
        ],
        out_specs=[
            pl.BlockSpec((1, 1, BE), lambda i: (i, 0, 0)),
            pl.BlockSpec((1, 1, BE), lambda i: (i, 0, 0)),
        ],
        out_shape=[
            jax.ShapeDtypeStruct((NB, 1, BE), jnp.float32),
            jax.ShapeDtypeStruct((NB, 1, BE), jnp.float32),
        ],
    )(W, b, feats)


def _tc_log_body(z_ref, v_ref):
    v_ref[...] = jnp.log(jnp.maximum(z_ref[...], 1e-30))


def _tc_log(z2d):
    return pl.pallas_call(
        _tc_log_body,
        out_shape=jax.ShapeDtypeStruct(z2d.shape, jnp.float32),
    )(z2d)


def _sc_body(NP, EPW, m_hbm, src_hbm, dst_hbm, bv_hbm, z_hbm, prob_hbm,
             m_v, pk_v, dst_v, z_v, z2_v, bv_v, part_v, red_v, zs_v,
             sp_part, sp_z):
    SL = NP // NS
    sid = lax.axis_index("s")
    base = sid * EPW

    pltpu.sync_copy(m_hbm.at[pl.ds(base, EPW)], m_v)
    pltpu.sync_copy(src_hbm.at[pl.ds(base, EPW)], pk_v)
    pltpu.sync_copy(dst_hbm.at[pl.ds(base, EPW)], dst_v)
    pltpu.sync_copy(bv_hbm, bv_v)
    pltpu.sync_copy(bv_hbm, z_v)

    @plsc.parallel_loop(0, EPW, L, unroll=8)
    def _(off):
        s = pk_v[pl.ds(off, L)]
        d = dst_v[pl.ds(off, L)]
        pk_v[pl.ds(off, L)] = lax.bitwise_or(lax.shift_left(s, 16), d)

    zero = jnp.zeros((L,), jnp.float32)

    def one_iter(z_src, z_dst):
        @plsc.parallel_loop(0, NP, L, unroll=8)
        def _(off):
            part_v[pl.ds(off, L)] = zero

        @plsc.parallel_loop(0, EPW, L, unroll=10)
        def _(off):
            pk = pk_v[pl.ds(off, L)]
            d = lax.bitwise_and(pk, 0xFFFF)
            s = lax.shift_right_logical(pk, 16)
            zg = plsc.load_gather(z_src, [d])
            msg = m_v[pl.ds(off, L)] * zg
            plsc.addupdate_scatter(part_v, [s], msg)

        pltpu.sync_copy(part_v, sp_part.at[sid])
        plsc.subcore_barrier()
        pltpu.sync_copy(sp_part.at[:, pl.ds(sid * SL, SL)], red_v)

        @plsc.parallel_loop(0, SL, L, unroll=4)
        def _(i):
            acc = bv_v[pl.ds(sid * SL + i, L)]
            for j in range(NS):
                acc = acc + red_v[j, pl.ds(i, L)]
            zs_v[pl.ds(i, L)] = acc

        pltpu.sync_copy(zs_v, sp_z.at[pl.ds(sid * SL, SL)])
        plsc.subcore_barrier()
        pltpu.sync_copy(sp_z, z_dst)

    def iter_cond(carry):
        t, conv = carry
        return jnp.logical_and(t < N_ITERS, conv == 0)

    def iter_body(carry):
        t, conv = carry
        one_iter(z_v, z2_v)
        one_iter(z2_v, z_v)

        def neq_body(i, acc):
            return acc + (z_v[pl.ds(i * L, L)] != z2_v[pl.ds(i * L, L)]
                          ).astype(jnp.float32)

        neq = lax.fori_loop(0, NP // L, neq_body, jnp.zeros((L,), jnp.float32))
        nz = jnp.max(neq)
        conv = jnp.where(nz == 0.0, 1, 0).astype(jnp.int32)
        return t + 2, conv

    lax.while_loop(iter_cond, iter_body, (jnp.int32(0), jnp.int32(0)))

    @plsc.parallel_loop(0, NP, L, unroll=8)
    def _(off):
        z_v[pl.ds(off, L)] = jnp.maximum(z_v[pl.ds(off, L)], 1e-30)
    pltpu.sync_copy(z_v.at[pl.ds(sid * SL, SL)], z_hbm.at[pl.ds(sid * SL, SL)])

    @plsc.parallel_loop(0, EPW, L, unroll=8)
    def _(off):
        pk = pk_v[pl.ds(off, L)]
        d = lax.bitwise_and(pk, 0xFFFF)
        s = lax.shift_right_logical(pk, 16)
        zd = plsc.load_gather(z_v, [d])
        zs = plsc.load_gather(z_v, [s])
        m_v[pl.ds(off, L)] = m_v[pl.ds(off, L)] * zd / zs
    pltpu.sync_copy(m_v, prob_hbm.at[pl.ds(base, EPW)])


def _sc_fixed_point(m, src, dst, bfull):
    E, = m.shape
    NP, = bfull.shape
    EPW = E // NS
    SL = NP // NS
    mesh = plsc.VectorSubcoreMesh(core_axis_name="c", subcore_axis_name="s",
                                  num_cores=1)
    f = pl.kernel(
        functools.partial(_sc_body, NP, EPW),
        out_type=[
            jax.ShapeDtypeStruct((NP,), jnp.float32),
            jax.ShapeDtypeStruct((E,), jnp.float32),
        ],
        mesh=mesh,
        compiler_params=pltpu.CompilerParams(needs_layout_passes=False),
        scratch_types=[
            pltpu.VMEM((EPW,), jnp.float32),
            pltpu.VMEM((EPW,), jnp.int32),
            pltpu.VMEM((EPW,), jnp.int32),
            pltpu.VMEM((NP,), jnp.float32),
            pltpu.VMEM((NP,), jnp.float32),
            pltpu.VMEM((NP,), jnp.float32),
            pltpu.VMEM((NP,), jnp.float32),
            pltpu.VMEM((NS, SL), jnp.float32),
            pltpu.VMEM((SL,), jnp.float32),
            pltpu.VMEM_SHARED((NS, NP), jnp.float32),
            pltpu.VMEM_SHARED((NP,), jnp.float32),
        ],
    )
    return f(m, src, dst, bfull)


def kernel(feats, dest_mask, edge_index, batch, n_nodes, W, b):
    E, F = feats.shape
    N = dest_mask.shape[0]
    NP = ((N + NS * L - 1) // (NS * L)) * (NS * L)
    if NP % 128 != 0:
        NP = ((NP + 127) // 128) * 128

    util2, m2 = _tc_util(feats, W, b)
    util = util2.reshape(E, 1)
    m = m2.reshape(E)

    src = edge_index[0]
    dst = edge_index[1]
    bfull = jnp.zeros((NP,), jnp.float32).at[:N].set(
        dest_mask.astype(jnp.float32))

    zc, prob = _sc_fixed_point(m, src, dst, bfull)

    value = _tc_log(zc.reshape(NP // 128, 128)).reshape(NP)[:N]
    return value, util, prob
